# baseline (speedup 1.0000x reference)
import jax
import jax.numpy as jnp
from jax.experimental import pallas as pl
from jax.experimental.pallas import tpu as pltpu

_TOPK = 10
_LAMBDA = 0.9
_NEG_SLOPE = 0.01


def _feats_body(ximg_ref, xtxt_ref, wimg_ref, bimg_ref, wtxt_ref, btxt_ref,
                fimg_ref, ftxt_ref):
    fi = jnp.dot(ximg_ref[...], wimg_ref[...],
                 preferred_element_type=jnp.float32) + bimg_ref[...]
    ft = jnp.dot(xtxt_ref[...], wtxt_ref[...],
                 preferred_element_type=jnp.float32) + btxt_ref[...]
    fi = fi * jax.lax.rsqrt(jnp.sum(fi * fi, axis=1, keepdims=True))
    ft = ft * jax.lax.rsqrt(jnp.sum(ft * ft, axis=1, keepdims=True))
    fimg_ref[...] = fi
    ftxt_ref[...] = ft


def _topk_thresh(s):
    bm, n = s.shape
    neg = jnp.full((bm, 128), -jnp.inf, jnp.float32)
    t1, t2, t3 = neg, neg, neg
    for j in range(n // 128):
        x = s[:, j * 128:(j + 1) * 128]
        d1 = jnp.minimum(t1, x)
        t1 = jnp.maximum(t1, x)
        d2 = jnp.minimum(t2, d1)
        t2 = jnp.maximum(t2, d1)
        t3 = jnp.maximum(t3, d2)
    v = jnp.concatenate([t1, t2, t3], axis=1)
    for _ in range(_TOPK - 1):
        mv = jnp.max(v, axis=1, keepdims=True)
        v = jnp.where(v >= mv, -jnp.inf, v)
    return jnp.max(v, axis=1, keepdims=True)


def _leaky(x):
    return jnp.where(x >= 0, x, _NEG_SLOPE * x)


def _gcn_block(adj_blk, ego_all, ego_blk, acc_blk, gcw, gcb, biw, bib):
    side = jnp.dot(adj_blk, ego_all, preferred_element_type=jnp.float32)
    sum_emb = _leaky(jnp.dot(side, gcw,
                             preferred_element_type=jnp.float32) + gcb)
    bi = ego_blk * side
    bi_emb = _leaky(jnp.dot(bi, biw,
                            preferred_element_type=jnp.float32) + bib)
    ego_new = sum_emb + bi_emb
    nrm = jnp.sqrt(jnp.sum(ego_new * ego_new, axis=1, keepdims=True))
    return ego_new, acc_blk + ego_new / jnp.maximum(nrm, 1e-12)


def _sims(fimg_blk, ftxt_blk, fimg_all, ftxt_all):
    dims = (((1,), (1,)), ((), ()))
    s_img = jax.lax.dot_general(fimg_blk, fimg_all, dims,
                                preferred_element_type=jnp.float32)
    s_txt = jax.lax.dot_general(ftxt_blk, ftxt_all, dims,
                                preferred_element_type=jnp.float32)
    return s_img, s_txt


def _mega1_body(adj_ref, ego_all_ref, ego_blk_ref, gcw_ref, gcb_ref,
                biw_ref, bib_ref,
                fimg_blk_ref, ftxt_blk_ref, fimg_all_ref, ftxt_all_ref, w_ref,
                imgo_ref, txto_ref, e_ref,
                ego_out_ref, acc_out_ref, a_ref, d_ref, ho_ref):
    ego_new, acc_new = _gcn_block(adj_ref[...], ego_all_ref[...],
                                  ego_blk_ref[...], ego_blk_ref[...],
                                  gcw_ref[...], gcb_ref[...], biw_ref[...],
                                  bib_ref[...])
    ego_out_ref[...] = ego_new
    acc_out_ref[...] = acc_new
    s_img, s_txt = _sims(fimg_blk_ref[...], ftxt_blk_ref[...],
                         fimg_all_ref[...], ftxt_all_ref[...])
    ti = _topk_thresh(s_img)
    tt = _topk_thresh(s_txt)
    a = (w_ref[0, 0] * jnp.where(s_img >= ti, s_img, 0.0)
         + w_ref[0, 1] * jnp.where(s_txt >= tt, s_txt, 0.0))
    a_ref[...] = a.astype(jnp.bfloat16)
    d_ref[...] = jnp.sum(a, axis=1, keepdims=True)
    orig = w_ref[0, 0] * imgo_ref[...] + w_ref[0, 1] * txto_ref[...]
    ho_ref[...] = jnp.dot(orig, e_ref[...], preferred_element_type=jnp.float32)


def _mega2_body(adj_ref, ego_all_ref, ego_blk_ref, acc_ref, gcw_ref, gcb_ref,
                biw_ref, bib_ref,
                a_ref, d_all_ref, d_blk_ref, e_ref, ho_ref,
                acc_out_ref, hn_ref):
    _, acc_new = _gcn_block(adj_ref[...], ego_all_ref[...], ego_blk_ref[...],
                            acc_ref[...], gcw_ref[...], gcb_ref[...],
                            biw_ref[...], bib_ref[...])
    acc_out_ref[...] = acc_new * (1.0 / 3.0)
    r_all = jax.lax.rsqrt(d_all_ref[...])
    dinv_all = jnp.where(jnp.isinf(r_all), 0.0, r_all)
    r_blk = jax.lax.rsqrt(d_blk_ref[...])
    dinv_blk = jnp.where(jnp.isinf(r_blk), 0.0, r_blk)
    u = e_ref[...] * dinv_all
    h_l = dinv_blk * jnp.dot(a_ref[...].astype(jnp.float32), u,
                             preferred_element_type=jnp.float32)
    h = (1.0 - _LAMBDA) * h_l + _LAMBDA * ho_ref[...]
    nrm = jnp.sqrt(jnp.sum(h * h, axis=1, keepdims=True))
    hn_ref[...] = h / jnp.maximum(nrm, 1e-12)


def kernel(adj, build_item_graph, user_emb, item_emb, image_feats_raw,
           text_feats_raw, image_trs_W, image_trs_b, text_trs_W, text_trs_b,
           modal_weight, GC_W0, GC_b0, GC_W1, GC_b1, Bi_W0, Bi_b0, Bi_W1,
           Bi_b1, image_original_adj, text_original_adj):
    n_items = item_emb.shape[0]
    n_users = user_emb.shape[0]
    n_all = n_items + n_users
    d_emb = item_emb.shape[1]

    w = jax.nn.softmax(modal_weight).reshape(1, 2)

    bm1 = 512
    fimg, ftxt = pl.pallas_call(
        _feats_body,
        grid=(n_items // bm1,),
        in_specs=[
            pl.BlockSpec((bm1, image_feats_raw.shape[1]), lambda i: (i, 0)),
            pl.BlockSpec((bm1, text_feats_raw.shape[1]), lambda i: (i, 0)),
            pl.BlockSpec(image_trs_W.shape, lambda i: (0, 0)),
            pl.BlockSpec((1, d_emb), lambda i: (0, 0)),
            pl.BlockSpec(text_trs_W.shape, lambda i: (0, 0)),
            pl.BlockSpec((1, d_emb), lambda i: (0, 0)),
        ],
        out_specs=[
            pl.BlockSpec((bm1, d_emb), lambda i: (i, 0)),
            pl.BlockSpec((bm1, d_emb), lambda i: (i, 0)),
        ],
        out_shape=[
            jax.ShapeDtypeStruct((n_items, d_emb), jnp.float32),
            jax.ShapeDtypeStruct((n_items, d_emb), jnp.float32),
        ],
        compiler_params=pltpu.CompilerParams(
            dimension_semantics=("parallel",)),
    )(image_feats_raw, text_feats_raw, image_trs_W,
      image_trs_b.reshape(1, d_emb), text_trs_W, text_trs_b.reshape(1, d_emb))

    bg = 256
    bs = 128
    bg2 = 512
    bs2 = 256
    ego0 = jnp.concatenate([user_emb, item_emb], axis=0)

    ego1, acc1, a_mat, d_vec, h_orig = pl.pallas_call(
        _mega1_body,
        grid=(n_all // bg,),
        in_specs=[
            pl.BlockSpec((bg, n_all), lambda i: (i, 0)),
            pl.BlockSpec((n_all, d_emb), lambda i: (0, 0)),
            pl.BlockSpec((bg, d_emb), lambda i: (i, 0)),
            pl.BlockSpec((d_emb, d_emb), lambda i: (0, 0)),
            pl.BlockSpec((1, d_emb), lambda i: (0, 0)),
            pl.BlockSpec((d_emb, d_emb), lambda i: (0, 0)),
            pl.BlockSpec((1, d_emb), lambda i: (0, 0)),
            pl.BlockSpec((bs, d_emb), lambda i: (i, 0)),
            pl.BlockSpec((bs, d_emb), lambda i: (i, 0)),
            pl.BlockSpec((n_items, d_emb), lambda i: (0, 0)),
            pl.BlockSpec((n_items, d_emb), lambda i: (0, 0)),
            pl.BlockSpec((1, 2), lambda i: (0, 0)),
            pl.BlockSpec((bs, n_items), lambda i: (i, 0)),
            pl.BlockSpec((bs, n_items), lambda i: (i, 0)),
            pl.BlockSpec((n_items, d_emb), lambda i: (0, 0)),
        ],
        out_specs=[
            pl.BlockSpec((bg, d_emb), lambda i: (i, 0)),
            pl.BlockSpec((bg, d_emb), lambda i: (i, 0)),
            pl.BlockSpec((bs, n_items), lambda i: (i, 0)),
            pl.BlockSpec((bs, 1), lambda i: (i, 0)),
            pl.BlockSpec((bs, d_emb), lambda i: (i, 0)),
        ],
        out_shape=[
            jax.ShapeDtypeStruct((n_all, d_emb), jnp.float32),
            jax.ShapeDtypeStruct((n_all, d_emb), jnp.float32),
            jax.ShapeDtypeStruct((n_items, n_items), jnp.bfloat16),
            jax.ShapeDtypeStruct((n_items, 1), jnp.float32),
            jax.ShapeDtypeStruct((n_items, d_emb), jnp.float32),
        ],
        compiler_params=pltpu.CompilerParams(
            dimension_semantics=("parallel",)),
    )(adj, ego0, ego0, GC_W0, GC_b0.reshape(1, d_emb), Bi_W0,
      Bi_b0.reshape(1, d_emb), fimg, ftxt, fimg, ftxt, w,
      image_original_adj, text_original_adj, item_emb)

    acc2, h_norm = pl.pallas_call(
        _mega2_body,
        grid=(n_all // bg2,),
        in_specs=[
            pl.BlockSpec((bg2, n_all), lambda i: (i, 0)),
            pl.BlockSpec((n_all, d_emb), lambda i: (0, 0)),
            pl.BlockSpec((bg2, d_emb), lambda i: (i, 0)),
            pl.BlockSpec((bg2, d_emb), lambda i: (i, 0)),
            pl.BlockSpec((d_emb, d_emb), lambda i: (0, 0)),
            pl.BlockSpec((1, d_emb), lambda i: (0, 0)),
            pl.BlockSpec((d_emb, d_emb), lambda i: (0, 0)),
            pl.BlockSpec((1, d_emb), lambda i: (0, 0)),
            pl.BlockSpec((bs2, n_items), lambda i: (i, 0)),
            pl.BlockSpec((n_items, 1), lambda i: (0, 0)),
            pl.BlockSpec((bs2, 1), lambda i: (i, 0)),
            pl.BlockSpec((n_items, d_emb), lambda i: (0, 0)),
            pl.BlockSpec((bs2, d_emb), lambda i: (i, 0)),
        ],
        out_specs=[
            pl.BlockSpec((bg2, d_emb), lambda i: (i, 0)),
            pl.BlockSpec((bs2, d_emb), lambda i: (i, 0)),
        ],
        out_shape=[
            jax.ShapeDtypeStruct((n_all, d_emb), jnp.float32),
            jax.ShapeDtypeStruct((n_items, d_emb), jnp.float32),
        ],
        compiler_params=pltpu.CompilerParams(
            dimension_semantics=("parallel",)),
    )(adj, ego1, ego1, acc1, GC_W1, GC_b1.reshape(1, d_emb), Bi_W1,
      Bi_b1.reshape(1, d_emb), a_mat, d_vec, d_vec, item_emb, h_orig)

    u_g = acc2[:n_users]
    i_g = acc2[n_users:] + h_norm
    return (u_g, i_g)
